# Initial kernel scaffold; baseline (speedup 1.0000x reference)
import jax
import jax.numpy as jnp
from jax.experimental import pallas as pl


def kernel(Y_user, Y_item, X_user, X_item, H_fwd, H_inv, w_fwd, w_inv, edge_src, edge_dst):
    raise NotImplementedError("write your pallas kernel here")


# speedup vs baseline: 3.5281x; 3.5281x over previous
import functools

import jax
import jax.numpy as jnp
from jax import lax
from jax.experimental import pallas as pl
from jax.experimental.pallas import tpu as pltpu
from jax.experimental.pallas import tpu_sc as plsc

N_USER = 10000
N_ITEM = 10000
E = 320000
K = 128
EPS = 0.1

NSUB = 16
CH = 128
E_SUB = E // NSUB
SLAB = 160
GRP = 32
NGRP = SLAB // GRP
NOUT = 10240
ACC_ROWS = NOUT + 128
DUMP = NOUT
ZB = ACC_ROWS // NSUB
WOUT = NOUT // NSUB

_mesh = plsc.VectorSubcoreMesh(core_axis_name="c", subcore_axis_name="s")


@functools.partial(
    pl.kernel,
    out_type=[
        jax.ShapeDtypeStruct((NOUT, K), jnp.float32),
        jax.ShapeDtypeStruct((NOUT, K), jnp.float32),
        jax.ShapeDtypeStruct((NOUT, K), jnp.float32),
        jax.ShapeDtypeStruct((NOUT, K), jnp.float32),
    ],
    mesh=_mesh,
    scratch_types=[
        pltpu.VMEM((GRP, CH), jnp.int32),
        pltpu.VMEM((GRP, CH), jnp.int32),
        pltpu.VMEM((GRP, CH), jnp.float32),
        pltpu.VMEM((CH, K), jnp.float32),
        pltpu.VMEM_SHARED((ACC_ROWS, K), jnp.float32),
    ],
)
def _sc_aggregate(Yu, Yi, src_g, src_s, dst_g, dst_s, wf2, wi2, zrows,
                  A_u, A_i, dg_u, dg_i,
                  idx_g, idx_s, w2, rows_v, acc):
    c = lax.axis_index("c")
    s = lax.axis_index("s")

    def run_dir(table, gidx2, sidx2, wsrc2, A_out, dg_out):
        r0 = s * SLAB
        zb = pl.multiple_of(s * ZB, 8)
        wb = pl.multiple_of(s * WOUT, 8)

        pltpu.sync_copy(zrows, acc.at[pl.ds(zb, ZB)])
        plsc.subcore_barrier()

        @pl.loop(0, NGRP)
        def _grp(g):
            gb = pl.multiple_of(r0 + g * GRP, 8)
            pltpu.sync_copy(gidx2.at[pl.ds(gb, GRP)], idx_g)
            pltpu.sync_copy(sidx2.at[pl.ds(gb, GRP)], idx_s)
            pltpu.sync_copy(wsrc2.at[pl.ds(gb, GRP)], w2)

            @pl.loop(0, GRP)
            def _chunk(j):
                pltpu.sync_copy(table.at[idx_g.at[j]], rows_v)

                @pl.loop(0, CH // 16)
                def _wm(v):
                    wvec = w2[j, pl.ds(v * 16, 16)]
                    for r16 in range(16):
                        r = v * 16 + r16
                        wr = wvec[r16]
                        for q in range(K // 16):
                            sl = pl.ds(q * 16, 16)
                            rows_v[r, sl] = rows_v[r, sl] * wr

                pltpu.sync_copy(rows_v, acc.at[idx_s.at[j]], add=True)

        plsc.subcore_barrier()
        pltpu.sync_copy(acc.at[pl.ds(wb, WOUT)], A_out.at[pl.ds(wb, WOUT)])
        plsc.subcore_barrier()

        pltpu.sync_copy(zrows, acc.at[pl.ds(zb, ZB)])

        @pl.loop(0, CH)
        def _fill(r):
            for q in range(K // 16):
                rows_v[r, pl.ds(q * 16, 16)] = jnp.full((16,), 1.0, jnp.float32)

        plsc.subcore_barrier()

        @pl.loop(0, NGRP)
        def _grp2(g):
            gb = pl.multiple_of(r0 + g * GRP, 8)
            pltpu.sync_copy(sidx2.at[pl.ds(gb, GRP)], idx_s)

            @pl.loop(0, GRP)
            def _chunk2(j):
                pltpu.sync_copy(rows_v, acc.at[idx_s.at[j]], add=True)

        plsc.subcore_barrier()
        pltpu.sync_copy(acc.at[pl.ds(wb, WOUT)], dg_out.at[pl.ds(wb, WOUT)])
        plsc.subcore_barrier()

    @pl.when(c == 0)
    def _():
        run_dir(Yi, dst_g, src_s, wi2, A_u, dg_u)

    @pl.when(c == 1)
    def _():
        run_dir(Yu, src_g, dst_s, wf2, A_i, dg_i)


def _tc_body(Xu, Yu, Au, du, Hf, Xi, Yi, Ai, di, Hi, out_u, out_i):
    scale = (EPS / K) * (EPS / K)
    ek = EPS / K
    dn = (((1,), (0,)), ((), ()))
    dnt = (((1,), (1,)), ((), ()))

    Hf_ = Hf[...]
    t = lax.dot_general(Yu[...], Hf_, dn, preferred_element_type=jnp.float32)
    t2 = lax.dot_general(t, Hf_, dnt, preferred_element_type=jnp.float32)
    ah = lax.dot_general(Au[...], Hf_, dnt, preferred_element_type=jnp.float32)
    out_u[...] = Xu[...] - du[:, 0:1] * t2 * scale + ah * ek

    Hi_ = Hi[...]
    ti = lax.dot_general(Yi[...], Hi_, dn, preferred_element_type=jnp.float32)
    ti2 = lax.dot_general(ti, Hi_, dnt, preferred_element_type=jnp.float32)
    ahi = lax.dot_general(Ai[...], Hi_, dnt, preferred_element_type=jnp.float32)
    out_i[...] = Xi[...] - di[:, 0:1] * ti2 * scale + ahi * ek


_TC_BLOCK = 1024


def _tc_combine(Xu, Yu, Au, du, Hf, Xi, Yi, Ai, di, Hi):
    grid = (NOUT // _TC_BLOCK,)
    row_spec = pl.BlockSpec((_TC_BLOCK, K), lambda i: (i, 0))
    h_spec = pl.BlockSpec((K, K), lambda i: (0, 0))
    return pl.pallas_call(
        _tc_body,
        grid=grid,
        in_specs=[row_spec, row_spec, row_spec, row_spec, h_spec,
                  row_spec, row_spec, row_spec, row_spec, h_spec],
        out_specs=[row_spec, row_spec],
        out_shape=[
            jax.ShapeDtypeStruct((NOUT, K), jnp.float32),
            jax.ShapeDtypeStruct((NOUT, K), jnp.float32),
        ],
    )(Xu, Yu, Au, du, Hf, Xi, Yi, Ai, di, Hi)


def _pad_edges(a, fill):
    a2 = a.reshape(NSUB, E_SUB)
    pad = jnp.full((NSUB, SLAB * CH - E_SUB), fill, a.dtype)
    return jnp.concatenate([a2, pad], axis=1).reshape(NSUB * SLAB, CH)


def kernel(Y_user, Y_item, X_user, X_item, H_fwd, H_inv, w_fwd, w_inv, edge_src, edge_dst):
    src_g = _pad_edges(edge_src, 0)
    src_s = _pad_edges(edge_src, DUMP)
    dst_g = _pad_edges(edge_dst, 0)
    dst_s = _pad_edges(edge_dst, DUMP)
    wf2 = _pad_edges(w_fwd.reshape(E), jnp.float32(0))
    wi2 = _pad_edges(w_inv.reshape(E), jnp.float32(0))
    zrows = jnp.zeros((ZB, K), jnp.float32)
    A_u, A_i, dg_u, dg_i = _sc_aggregate(
        Y_user, Y_item, src_g, src_s, dst_g, dst_s, wf2, wi2, zrows)
    pad = ((0, NOUT - N_USER), (0, 0))
    ret_u, ret_i = _tc_combine(
        jnp.pad(X_user, pad), jnp.pad(Y_user, pad), A_u, dg_u, H_fwd,
        jnp.pad(X_item, pad), jnp.pad(Y_item, pad), A_i, dg_i, H_inv)
    return ret_u[:N_USER], ret_i[:N_ITEM]
